# baseline (speedup 1.0000x reference)
import math

import jax
import jax.numpy as jnp
from jax import lax
from jax.experimental import pallas as pl
from jax.experimental.pallas import tpu as pltpu
from jax.experimental.pallas import tpu_sc as plsc

_V = 1000000
_D = 64
_SCALE = math.sqrt(_D)
_NC = 2
_NS = 16
_NW = _NC * _NS
_NFULL = _V // 128
_TAIL = _NFULL * 128
_SROWS = _TAIL // 2 + 64


def _transpose_body(nat_hbm, tail_hbm, scr_hbm, inb, tb, sem):
    wid = lax.axis_index("s") * _NC + lax.axis_index("c")
    row_idx = [lax.iota(jnp.int32, 16) + dg * 16 for dg in range(4)]

    def do_block(c):
        pltpu.sync_copy(nat_hbm.at[:, pl.ds(c * 128, 128)], inb)

        def tpose(q, carry):
            for half in range(2):
                v = 2 * q + half
                col = jnp.full((16,), v, dtype=jnp.int32)
                for dg in range(4):
                    vec = plsc.load_gather(inb, [row_idx[dg], col])
                    tb[q, pl.ds(half * 64 + dg * 16, 16)] = vec
            return carry

        lax.fori_loop(0, 64, tpose, 0, unroll=2)
        pltpu.sync_copy(tb, scr_hbm.at[pl.ds(c * 64, 64)])

    def blk_loop(t, carry):
        c = wid + t * _NW

        @pl.when(c < _NFULL)
        def _():
            do_block(c)

        return carry

    lax.fori_loop(0, (_NFULL + _NW - 1) // _NW, blk_loop, 0)

    @pl.when(wid == 0)
    def _():
        pltpu.sync_copy(tail_hbm, inb)
        pltpu.sync_copy(inb, scr_hbm.at[pl.ds(_TAIL // 2, 64)])


def _gather_body(tbl_hbm, xf_hbm, out_hbm, idxv, idxg, rows, outv, gsem,
                 wsem):
    wid = lax.axis_index("s") * _NC + lax.axis_index("c")
    ngrp = 2560 // _NW
    iotas = [lax.iota(jnp.int32, 16) + dg * 16 for dg in range(4)]

    def do_group(i, carry):
        g = wid * ngrp + i
        s = g // 128
        cb = g % 128
        base = s * 16384 + cb * 128
        pltpu.sync_copy(xf_hbm.at[pl.ds(base, 128)], idxv)

        def remap(j, c2):
            vv = idxv[pl.ds(j * 16, 16)]
            idxg[pl.ds(j * 16, 16)] = jnp.where(vv < _TAIL, vv,
                                                2 * vv - _TAIL)
            return c2

        lax.fori_loop(0, 8, remap, 0, unroll=8)
        pltpu.async_copy(tbl_hbm.at[idxg], rows, gsem).wait()

        def tpose(lb, c2):
            col = jnp.full((16,), lb, dtype=jnp.int32)
            for dg in range(4):
                vec = rows[lb, pl.ds(dg * 16, 16)] * _SCALE
                plsc.store_scatter(outv, [iotas[dg], col], vec)
            return c2

        lax.fori_loop(0, 128, tpose, 0, unroll=2)
        for gd in range(8):
            pltpu.async_copy(outv.at[pl.ds(gd * 8, 8)],
                             out_hbm.at[s, gd, cb], wsem)
        for gd in range(8):
            pltpu.make_async_copy(outv.at[pl.ds(gd * 8, 8)],
                                  out_hbm.at[s, gd, cb], wsem).wait()
        return carry

    lax.fori_loop(0, ngrp, do_group, 0)


def kernel(x, emb_weight):
    natv = jnp.transpose(emb_weight)
    xf = jnp.reshape(jnp.transpose(x), (327680,)).astype(jnp.int32)
    tailp = jnp.concatenate(
        [emb_weight[_TAIL:, :],
         jnp.zeros((_V - _TAIL, _D), jnp.float32)], axis=1)

    mesh = plsc.VectorSubcoreMesh(core_axis_name="c", subcore_axis_name="s")
    k1 = pl.kernel(
        _transpose_body,
        mesh=mesh,
        out_type=jax.ShapeDtypeStruct((_SROWS, 128), jnp.float32),
        scratch_types=[
            pltpu.VMEM((64, 128), jnp.float32),
            pltpu.VMEM((64, 128), jnp.float32),
            pltpu.SemaphoreType.DMA,
        ],
        compiler_params=pltpu.CompilerParams(
            use_tc_tiling_on_sc=True, needs_layout_passes=False),
    )
    scratch = k1(natv, tailp)
    scr_view = jnp.reshape(scratch, (2 * _SROWS, _D))

    k2 = pl.kernel(
        _gather_body,
        mesh=mesh,
        out_type=jax.ShapeDtypeStruct((20, 8, 128, 8, 128), jnp.float32),
        scratch_types=[
            pltpu.VMEM((128,), jnp.int32),
            pltpu.VMEM((128,), jnp.int32),
            pltpu.VMEM((128, _D), jnp.float32),
            pltpu.VMEM((_D, 128), jnp.float32),
            pltpu.SemaphoreType.DMA,
            pltpu.SemaphoreType.DMA,
        ],
        compiler_params=pltpu.CompilerParams(
            use_tc_tiling_on_sc=False, needs_layout_passes=False),
    )
    out5 = k2(scr_view, xf)
    return jnp.reshape(jnp.transpose(out5, (2, 4, 0, 1, 3)), (16384, 20, 64))


# speedup vs baseline: 1.7817x; 1.7817x over previous
import math

import jax
import jax.numpy as jnp
from jax import lax
from jax.experimental import pallas as pl
from jax.experimental.pallas import tpu as pltpu
from jax.experimental.pallas import tpu_sc as plsc

_V = 1000000
_D = 64
_SCALE = math.sqrt(_D)
_NC = 2
_NS = 16
_NW = _NC * _NS
_NFULL = _V // 128
_TAIL = _NFULL * 128
_SROWS = _TAIL // 2 + 64
_SKEW = 137


def _transpose_body(nat_hbm, tail_hbm, scr_hbm, inb0, inb1, tb1, tbc,
                    isem0, isem1, osem):
    wid = lax.axis_index("s") * _NC + lax.axis_index("c")
    iota = lax.iota(jnp.int32, 16)
    vbase = [(vg * 8 + (iota >> 1)) * _SKEW + (iota & 1) * 72
             for vg in range(8)]
    inbs = (inb0, inb1)
    isems = (isem0, isem1)
    nt = (_NFULL + _NW - 1) // _NW

    def cof(t):
        return (wid + t * _NW) * 128

    pltpu.async_copy(nat_hbm.at[:, pl.ds(cof(0), 128)], inb0, isem0)

    def do_block(t, buf):
        c = wid + t * _NW
        inb = inbs[buf]

        @pl.when(c + _NW < _NFULL)
        def _():
            pltpu.async_copy(nat_hbm.at[:, pl.ds(cof(t + 1), 128)],
                             inbs[1 - buf], isems[1 - buf])

        pltpu.make_async_copy(nat_hbm.at[:, pl.ds(cof(t), 128)], inb,
                              isems[buf]).wait()

        def dloop(d, c2):
            for vg in range(8):
                vec = inb[d, pl.ds(vg * 16, 16)]
                plsc.store_scatter(tb1, [vbase[vg] + d], vec)
            return c2

        lax.fori_loop(0, 64, dloop, 0, unroll=2)

        def qloop(q, c2):
            for h in range(2):
                for dg in range(4):
                    tbc[q, pl.ds(h * 64 + dg * 16, 16)] = tb1[
                        pl.ds(q * _SKEW + h * 72 + dg * 16, 16)]
            return c2

        lax.fori_loop(0, 64, qloop, 0, unroll=2)
        pltpu.sync_copy(tbc, scr_hbm.at[pl.ds(c * 64, 64)])

    def blk_loop(t0, carry):
        for b in range(2):
            t = t0 * 2 + b

            @pl.when(wid + t * _NW < _NFULL)
            def _():
                do_block(t, b)

        return carry

    lax.fori_loop(0, (nt + 1) // 2, blk_loop, 0)

    @pl.when(wid == 0)
    def _():
        pltpu.sync_copy(tail_hbm, inb0)
        pltpu.sync_copy(inb0, scr_hbm.at[pl.ds(_TAIL // 2, 64)])


def _gather_body(tbl_hbm, xf_hbm, out_hbm, idxv, idxg0, idxg1, rows0, rows1,
                 out1, outc, gsem0, gsem1, wsem):
    wid = lax.axis_index("s") * _NC + lax.axis_index("c")
    ngrp = 2560 // _NW
    iota = lax.iota(jnp.int32, 16)
    dgbase = [(dg * 16 + iota) * _SKEW for dg in range(4)]
    idxgs = (idxg0, idxg1)
    rowss = (rows0, rows1)
    gsems = (gsem0, gsem1)

    def stage(i, buf):
        g = wid * ngrp + i
        base = (g // 128) * 16384 + (g % 128) * 128
        pltpu.sync_copy(xf_hbm.at[pl.ds(base, 128)], idxv)

        def remap(j, c2):
            vv = idxv[pl.ds(j * 16, 16)]
            idxgs[buf][pl.ds(j * 16, 16)] = jnp.where(vv < _TAIL, vv,
                                                      2 * vv - _TAIL)
            return c2

        lax.fori_loop(0, 8, remap, 0, unroll=8)
        pltpu.async_copy(tbl_hbm.at[idxgs[buf]], rowss[buf], gsems[buf])

    stage(0, 0)

    def do_group(i, buf):
        g = wid * ngrp + i
        s = g // 128
        cb = g % 128
        pltpu.make_async_copy(tbl_hbm.at[idxgs[buf]], rowss[buf],
                              gsems[buf]).wait()

        @pl.when(i + 1 < ngrp)
        def _():
            stage(i + 1, 1 - buf)

        rows = rowss[buf]

        def lbloop(lb, c2):
            for dg in range(4):
                vec = rows[lb, pl.ds(dg * 16, 16)] * _SCALE
                plsc.store_scatter(out1, [dgbase[dg] + lb], vec)
            return c2

        lax.fori_loop(0, 128, lbloop, 0, unroll=2)

        def dloop(d, c2):
            for lbg in range(8):
                outc[d, pl.ds(lbg * 16, 16)] = out1[
                    pl.ds(d * _SKEW + lbg * 16, 16)]
            return c2

        lax.fori_loop(0, 64, dloop, 0, unroll=2)
        for gd in range(8):
            pltpu.async_copy(outc.at[pl.ds(gd * 8, 8)],
                             out_hbm.at[s, gd, cb], wsem)
        for gd in range(8):
            pltpu.make_async_copy(outc.at[pl.ds(gd * 8, 8)],
                                  out_hbm.at[s, gd, cb], wsem).wait()

    def grp_loop(i0, carry):
        for b in range(2):
            do_group(i0 * 2 + b, b)
        return carry

    lax.fori_loop(0, ngrp // 2, grp_loop, 0)


def kernel(x, emb_weight):
    natv = jnp.transpose(emb_weight)
    xf = jnp.reshape(jnp.transpose(x), (327680,)).astype(jnp.int32)
    tailp = jnp.concatenate(
        [emb_weight[_TAIL:, :],
         jnp.zeros((_V - _TAIL, _D), jnp.float32)], axis=1)

    mesh = plsc.VectorSubcoreMesh(core_axis_name="c", subcore_axis_name="s")
    k1 = pl.kernel(
        _transpose_body,
        mesh=mesh,
        out_type=jax.ShapeDtypeStruct((_SROWS, 128), jnp.float32),
        scratch_types=[
            pltpu.VMEM((64, 128), jnp.float32),
            pltpu.VMEM((64, 128), jnp.float32),
            pltpu.VMEM((64 * _SKEW,), jnp.float32),
            pltpu.VMEM((64, 128), jnp.float32),
            pltpu.SemaphoreType.DMA,
            pltpu.SemaphoreType.DMA,
            pltpu.SemaphoreType.DMA,
        ],
        compiler_params=pltpu.CompilerParams(
            use_tc_tiling_on_sc=True, needs_layout_passes=False),
    )
    scratch = k1(natv, tailp)
    scr_view = jnp.reshape(scratch, (2 * _SROWS, _D))

    k2 = pl.kernel(
        _gather_body,
        mesh=mesh,
        out_type=jax.ShapeDtypeStruct((20, 8, 128, 8, 128), jnp.float32),
        scratch_types=[
            pltpu.VMEM((128,), jnp.int32),
            pltpu.VMEM((128,), jnp.int32),
            pltpu.VMEM((128,), jnp.int32),
            pltpu.VMEM((128, _D), jnp.float32),
            pltpu.VMEM((128, _D), jnp.float32),
            pltpu.VMEM((64 * _SKEW,), jnp.float32),
            pltpu.VMEM((_D, 128), jnp.float32),
            pltpu.SemaphoreType.DMA,
            pltpu.SemaphoreType.DMA,
            pltpu.SemaphoreType.DMA,
        ],
        compiler_params=pltpu.CompilerParams(
            use_tc_tiling_on_sc=False, needs_layout_passes=False),
    )
    out5 = k2(scr_view, xf)
    return jnp.reshape(jnp.transpose(out5, (2, 4, 0, 1, 3)), (16384, 20, 64))


# speedup vs baseline: 1.9990x; 1.1219x over previous
import math

import jax
import jax.numpy as jnp
from jax import lax
from jax.experimental import pallas as pl
from jax.experimental.pallas import tpu as pltpu
from jax.experimental.pallas import tpu_sc as plsc

_V = 1000000
_D = 64
_SCALE = math.sqrt(_D)
_NC = 2
_NS = 16
_NW = _NC * _NS
_NFULL = _V // 128
_TAIL = _NFULL * 128
_SROWS = _TAIL // 2 + 64
_SKEW = 137


def _transpose_body(nat_hbm, tail_hbm, scr_hbm, inb0, inb1, tb1, tbc,
                    isem0, isem1, osem):
    wid = lax.axis_index("s") * _NC + lax.axis_index("c")
    iota = lax.iota(jnp.int32, 16)
    vbase = [(vg * 8 + (iota >> 1)) * _SKEW + (iota & 1) * 72
             for vg in range(8)]
    inbs = (inb0, inb1)
    isems = (isem0, isem1)
    nt = (_NFULL + _NW - 1) // _NW

    def cof(t):
        return (wid + t * _NW) * 128

    pltpu.async_copy(nat_hbm.at[:, pl.ds(cof(0), 128)], inb0, isem0)

    def do_block(t, buf):
        c = wid + t * _NW
        inb = inbs[buf]

        @pl.when(c + _NW < _NFULL)
        def _():
            pltpu.async_copy(nat_hbm.at[:, pl.ds(cof(t + 1), 128)],
                             inbs[1 - buf], isems[1 - buf])

        pltpu.make_async_copy(nat_hbm.at[:, pl.ds(cof(t), 128)], inb,
                              isems[buf]).wait()

        def dloop(d, c2):
            for vg in range(8):
                vec = inb[d, pl.ds(vg * 16, 16)]
                plsc.store_scatter(tb1, [vbase[vg] + d], vec)
            return c2

        lax.fori_loop(0, 64, dloop, 0, unroll=4)

        def qloop(q, c2):
            for h in range(2):
                for dg in range(4):
                    tbc[q, pl.ds(h * 64 + dg * 16, 16)] = tb1[
                        pl.ds(q * _SKEW + h * 72 + dg * 16, 16)]
            return c2

        lax.fori_loop(0, 64, qloop, 0, unroll=4)
        pltpu.sync_copy(tbc, scr_hbm.at[pl.ds(c * 64, 64)])

    def blk_loop(t0, carry):
        for b in range(2):
            t = t0 * 2 + b

            @pl.when(wid + t * _NW < _NFULL)
            def _():
                do_block(t, b)

        return carry

    lax.fori_loop(0, (nt + 1) // 2, blk_loop, 0)

    @pl.when(wid == 0)
    def _():
        pltpu.sync_copy(tail_hbm, inb0)
        pltpu.sync_copy(inb0, scr_hbm.at[pl.ds(_TAIL // 2, 64)])


def _gather_body(tbl_hbm, xf_hbm, out_hbm, idxv, idxg0, idxg1, rows0, rows1,
                 outc, gsem0, gsem1, wsem):
    wid = lax.axis_index("s") * _NC + lax.axis_index("c")
    ngrp = 2560 // _NW
    iota = lax.iota(jnp.int32, 16)
    dgrow = [dg * 16 + iota for dg in range(4)]
    idxgs = (idxg0, idxg1)
    rowss = (rows0, rows1)
    gsems = (gsem0, gsem1)

    def stage(i, buf):
        g = wid * ngrp + i
        base = (g // 128) * 16384 + (g % 128) * 128
        pltpu.sync_copy(xf_hbm.at[pl.ds(base, 128)], idxv)

        def remap(j, c2):
            vv = idxv[pl.ds(j * 16, 16)]
            idxgs[buf][pl.ds(j * 16, 16)] = jnp.where(vv < _TAIL, vv,
                                                      2 * vv - _TAIL)
            return c2

        lax.fori_loop(0, 8, remap, 0, unroll=8)
        pltpu.async_copy(tbl_hbm.at[idxgs[buf]], rowss[buf], gsems[buf])

    stage(0, 0)

    def do_group(i, buf):
        g = wid * ngrp + i
        s = g // 128
        cb = g % 128
        pltpu.make_async_copy(tbl_hbm.at[idxgs[buf]], rowss[buf],
                              gsems[buf]).wait()

        @pl.when(i + 1 < ngrp)
        def _():
            stage(i + 1, 1 - buf)

        rows = rowss[buf]

        def lbloop(lb, c2):
            col = jnp.full((16,), lb, dtype=jnp.int32)
            for dg in range(4):
                vec = rows[lb, pl.ds(dg * 16, 16)] * _SCALE
                plsc.store_scatter(outc, [dgrow[dg], col], vec)
            return c2

        lax.fori_loop(0, 128, lbloop, 0, unroll=4)
        for gd in range(8):
            pltpu.async_copy(outc.at[pl.ds(gd * 8, 8), pl.ds(0, 128)],
                             out_hbm.at[s, gd, cb], wsem)
        for gd in range(8):
            pltpu.make_async_copy(outc.at[pl.ds(gd * 8, 8), pl.ds(0, 128)],
                                  out_hbm.at[s, gd, cb], wsem).wait()

    def grp_loop(i0, carry):
        for b in range(2):
            do_group(i0 * 2 + b, b)
        return carry

    lax.fori_loop(0, ngrp // 2, grp_loop, 0)


def kernel(x, emb_weight):
    natv = jnp.transpose(emb_weight)
    xf = jnp.reshape(jnp.transpose(x), (327680,)).astype(jnp.int32)
    tailp = jnp.concatenate(
        [emb_weight[_TAIL:, :],
         jnp.zeros((_V - _TAIL, _D), jnp.float32)], axis=1)

    mesh = plsc.VectorSubcoreMesh(core_axis_name="c", subcore_axis_name="s")
    k1 = pl.kernel(
        _transpose_body,
        mesh=mesh,
        out_type=jax.ShapeDtypeStruct((_SROWS, 128), jnp.float32),
        scratch_types=[
            pltpu.VMEM((64, 128), jnp.float32),
            pltpu.VMEM((64, 128), jnp.float32),
            pltpu.VMEM((64 * _SKEW,), jnp.float32),
            pltpu.VMEM((64, 128), jnp.float32),
            pltpu.SemaphoreType.DMA,
            pltpu.SemaphoreType.DMA,
            pltpu.SemaphoreType.DMA,
        ],
        compiler_params=pltpu.CompilerParams(
            use_tc_tiling_on_sc=True, needs_layout_passes=False),
    )
    scratch = k1(natv, tailp)
    scr_view = jnp.reshape(scratch, (2 * _SROWS, _D))

    k2 = pl.kernel(
        _gather_body,
        mesh=mesh,
        out_type=jax.ShapeDtypeStruct((20, 8, 128, 8, 128), jnp.float32),
        scratch_types=[
            pltpu.VMEM((128,), jnp.int32),
            pltpu.VMEM((128,), jnp.int32),
            pltpu.VMEM((128,), jnp.int32),
            pltpu.VMEM((128, _D), jnp.float32),
            pltpu.VMEM((128, _D), jnp.float32),
            pltpu.VMEM((_D, 136), jnp.float32),
            pltpu.SemaphoreType.DMA,
            pltpu.SemaphoreType.DMA,
            pltpu.SemaphoreType.DMA,
        ],
        compiler_params=pltpu.CompilerParams(
            use_tc_tiling_on_sc=False, needs_layout_passes=False),
    )
    out5 = k2(scr_view, xf)
    return jnp.reshape(jnp.transpose(out5, (2, 4, 0, 1, 3)), (16384, 20, 64))


# speedup vs baseline: 5.7257x; 2.8644x over previous
import math

import jax
import jax.numpy as jnp
from jax import lax
from jax.experimental import pallas as pl
from jax.experimental.pallas import tpu as pltpu
from jax.experimental.pallas import tpu_sc as plsc

_V = 1000000
_D = 64
_SCALE = math.sqrt(_D)
_NC = 2
_NS = 16
_NW = _NC * _NS
_NFULL = _V // 128
_TAIL = _NFULL * 128
_SROWS = _TAIL // 2 + 64
_SKEW = 137


def _transpose_body(nat_hbm, tail_hbm, scr_hbm, inb0, inb1, tb1, tbc,
                    isem0, isem1, osem):
    wid = lax.axis_index("s") * _NC + lax.axis_index("c")
    iota = lax.iota(jnp.int32, 16)
    vbase = [(vg * 8 + (iota >> 1)) * _SKEW + (iota & 1) * 72
             for vg in range(8)]
    inbs = (inb0, inb1)
    isems = (isem0, isem1)
    nt = (_NFULL + _NW - 1) // _NW

    def cof(t):
        return (wid + t * _NW) * 128

    pltpu.async_copy(nat_hbm.at[:, pl.ds(cof(0), 128)], inb0, isem0)

    def do_block(t, buf):
        c = wid + t * _NW
        inb = inbs[buf]

        @pl.when(c + _NW < _NFULL)
        def _():
            pltpu.async_copy(nat_hbm.at[:, pl.ds(cof(t + 1), 128)],
                             inbs[1 - buf], isems[1 - buf])

        pltpu.make_async_copy(nat_hbm.at[:, pl.ds(cof(t), 128)], inb,
                              isems[buf]).wait()

        @plsc.parallel_loop(0, 64, unroll=4)
        def _(d):
            for vg in range(8):
                vec = inb[d, pl.ds(vg * 16, 16)]
                plsc.store_scatter(tb1, [vbase[vg] + d], vec)

        @plsc.parallel_loop(0, 64, unroll=4)
        def _(q):
            for h in range(2):
                for dg in range(4):
                    tbc[q, pl.ds(h * 64 + dg * 16, 16)] = tb1[
                        pl.ds(q * _SKEW + h * 72 + dg * 16, 16)]
        pltpu.sync_copy(tbc, scr_hbm.at[pl.ds(c * 64, 64)])

    def blk_loop(t0, carry):
        for b in range(2):
            t = t0 * 2 + b

            @pl.when(wid + t * _NW < _NFULL)
            def _():
                do_block(t, b)

        return carry

    lax.fori_loop(0, (nt + 1) // 2, blk_loop, 0)

    @pl.when(wid == 0)
    def _():
        pltpu.sync_copy(tail_hbm, inb0)
        pltpu.sync_copy(inb0, scr_hbm.at[pl.ds(_TAIL // 2, 64)])


def _gather_body(tbl_hbm, xf_hbm, out_hbm, idxv, idxg0, idxg1, rows0, rows1,
                 outc, gsem0, gsem1, wsem):
    wid = lax.axis_index("s") * _NC + lax.axis_index("c")
    ngrp = 2560 // _NW
    iota = lax.iota(jnp.int32, 16)
    dgrow = [dg * 16 + iota for dg in range(4)]
    idxgs = (idxg0, idxg1)
    rowss = (rows0, rows1)
    gsems = (gsem0, gsem1)

    def stage(i, buf):
        g = wid * ngrp + i
        base = (g // 128) * 16384 + (g % 128) * 128
        pltpu.sync_copy(xf_hbm.at[pl.ds(base, 128)], idxv)

        def remap(j, c2):
            vv = idxv[pl.ds(j * 16, 16)]
            idxgs[buf][pl.ds(j * 16, 16)] = jnp.where(vv < _TAIL, vv,
                                                      2 * vv - _TAIL)
            return c2

        lax.fori_loop(0, 8, remap, 0, unroll=8)
        pltpu.async_copy(tbl_hbm.at[idxgs[buf]], rowss[buf], gsems[buf])

    stage(0, 0)

    def do_group(i, buf):
        g = wid * ngrp + i
        s = g // 128
        cb = g % 128
        pltpu.make_async_copy(tbl_hbm.at[idxgs[buf]], rowss[buf],
                              gsems[buf]).wait()

        @pl.when(i + 1 < ngrp)
        def _():
            stage(i + 1, 1 - buf)

        rows = rowss[buf]

        @plsc.parallel_loop(0, 128, unroll=4)
        def _(lb):
            col = jnp.full((16,), lb, dtype=jnp.int32)
            for dg in range(4):
                vec = rows[lb, pl.ds(dg * 16, 16)] * _SCALE
                plsc.store_scatter(outc, [dgrow[dg], col], vec)
        for gd in range(8):
            pltpu.async_copy(outc.at[pl.ds(gd * 8, 8), pl.ds(0, 128)],
                             out_hbm.at[s, gd, cb], wsem)
        for gd in range(8):
            pltpu.make_async_copy(outc.at[pl.ds(gd * 8, 8), pl.ds(0, 128)],
                                  out_hbm.at[s, gd, cb], wsem).wait()

    def grp_loop(i0, carry):
        for b in range(2):
            do_group(i0 * 2 + b, b)
        return carry

    lax.fori_loop(0, ngrp // 2, grp_loop, 0)


def kernel(x, emb_weight):
    natv = jnp.transpose(emb_weight)
    xf = jnp.reshape(jnp.transpose(x), (327680,)).astype(jnp.int32)
    tailp = jnp.concatenate(
        [emb_weight[_TAIL:, :],
         jnp.zeros((_V - _TAIL, _D), jnp.float32)], axis=1)

    mesh = plsc.VectorSubcoreMesh(core_axis_name="c", subcore_axis_name="s")
    k1 = pl.kernel(
        _transpose_body,
        mesh=mesh,
        out_type=jax.ShapeDtypeStruct((_SROWS, 128), jnp.float32),
        scratch_types=[
            pltpu.VMEM((64, 128), jnp.float32),
            pltpu.VMEM((64, 128), jnp.float32),
            pltpu.VMEM((64 * _SKEW,), jnp.float32),
            pltpu.VMEM((64, 128), jnp.float32),
            pltpu.SemaphoreType.DMA,
            pltpu.SemaphoreType.DMA,
            pltpu.SemaphoreType.DMA,
        ],
        compiler_params=pltpu.CompilerParams(
            use_tc_tiling_on_sc=True, needs_layout_passes=False),
    )
    scratch = k1(natv, tailp)
    scr_view = jnp.reshape(scratch, (2 * _SROWS, _D))

    k2 = pl.kernel(
        _gather_body,
        mesh=mesh,
        out_type=jax.ShapeDtypeStruct((20, 8, 128, 8, 128), jnp.float32),
        scratch_types=[
            pltpu.VMEM((128,), jnp.int32),
            pltpu.VMEM((128,), jnp.int32),
            pltpu.VMEM((128,), jnp.int32),
            pltpu.VMEM((128, _D), jnp.float32),
            pltpu.VMEM((128, _D), jnp.float32),
            pltpu.VMEM((_D, 136), jnp.float32),
            pltpu.SemaphoreType.DMA,
            pltpu.SemaphoreType.DMA,
            pltpu.SemaphoreType.DMA,
        ],
        compiler_params=pltpu.CompilerParams(
            use_tc_tiling_on_sc=False, needs_layout_passes=False),
    )
    out5 = k2(scr_view, xf)
    return jnp.reshape(jnp.transpose(out5, (2, 4, 0, 1, 3)), (16384, 20, 64))


# speedup vs baseline: 6.2566x; 1.0927x over previous
import math

import jax
import jax.numpy as jnp
from jax import lax
from jax.experimental import pallas as pl
from jax.experimental.pallas import tpu as pltpu
from jax.experimental.pallas import tpu_sc as plsc

_V = 1000000
_D = 64
_SCALE = math.sqrt(_D)
_NC = 2
_NS = 16
_NW = _NC * _NS
_NFULL = _V // 128
_TAIL = _NFULL * 128
_SROWS = _TAIL // 2 + 64
_SKEW = 137


def _transpose_body(nat_hbm, tail_hbm, scr_hbm, inb0, inb1, tb1, tbc0, tbc1,
                    isem0, isem1, osem0, osem1):
    wid = lax.axis_index("s") * _NC + lax.axis_index("c")
    iota = lax.iota(jnp.int32, 16)
    vbase = [(vg * 8 + (iota >> 1)) * _SKEW + (iota & 1) * 72
             for vg in range(8)]
    inbs = (inb0, inb1)
    isems = (isem0, isem1)
    tbcs = (tbc0, tbc1)
    osems = (osem0, osem1)
    nt = (_NFULL + _NW - 1) // _NW

    def cof(t):
        return (wid + t * _NW) * 128

    pltpu.async_copy(nat_hbm.at[:, pl.ds(cof(0), 128)], inb0, isem0)

    def do_block(t, buf):
        c = wid + t * _NW
        inb = inbs[buf]

        @pl.when(c + _NW < _NFULL)
        def _():
            pltpu.async_copy(nat_hbm.at[:, pl.ds(cof(t + 1), 128)],
                             inbs[1 - buf], isems[1 - buf])

        pltpu.make_async_copy(nat_hbm.at[:, pl.ds(cof(t), 128)], inb,
                              isems[buf]).wait()

        @plsc.parallel_loop(0, 64, unroll=4)
        def _(d):
            for vg in range(8):
                vec = inb[d, pl.ds(vg * 16, 16)]
                plsc.store_scatter(tb1, [vbase[vg] + d], vec)

        @pl.when(t >= 2)
        def _():
            cp = c - 2 * _NW
            pltpu.make_async_copy(tbcs[buf],
                                  scr_hbm.at[pl.ds(cp * 64, 64)],
                                  osems[buf]).wait()

        @plsc.parallel_loop(0, 64, unroll=4)
        def _(q):
            for h in range(2):
                for dg in range(4):
                    tbcs[buf][q, pl.ds(h * 64 + dg * 16, 16)] = tb1[
                        pl.ds(q * _SKEW + h * 72 + dg * 16, 16)]
        pltpu.async_copy(tbcs[buf], scr_hbm.at[pl.ds(c * 64, 64)],
                         osems[buf])

    def blk_loop(t0, carry):
        for b in range(2):
            t = t0 * 2 + b

            @pl.when(wid + t * _NW < _NFULL)
            def _():
                do_block(t, b)

        return carry

    lax.fori_loop(0, (nt + 1) // 2, blk_loop, 0)

    tmax = (_NFULL - 1 - wid) // _NW
    c1 = wid + tmax * _NW
    for b in range(2):
        cb = jnp.where(tmax % 2 == b, c1, c1 - _NW)
        pltpu.make_async_copy(tbcs[b], scr_hbm.at[pl.ds(cb * 64, 64)],
                              osems[b]).wait()

    @pl.when(wid == 0)
    def _():
        pltpu.sync_copy(tail_hbm, inb0)
        pltpu.sync_copy(inb0, scr_hbm.at[pl.ds(_TAIL // 2, 64)])


def _gather_body(tbl_hbm, xf_hbm, out_hbm, idxv, idxg0, idxg1, rows0, rows1,
                 outc, gsem0, gsem1, wsem):
    wid = lax.axis_index("s") * _NC + lax.axis_index("c")
    ngrp = 2560 // _NW
    iota = lax.iota(jnp.int32, 16)
    dgrow = [dg * 16 + iota for dg in range(4)]
    idxgs = (idxg0, idxg1)
    rowss = (rows0, rows1)
    gsems = (gsem0, gsem1)

    def stage(i, buf):
        g = wid * ngrp + i
        base = (g // 128) * 16384 + (g % 128) * 128
        pltpu.sync_copy(xf_hbm.at[pl.ds(base, 128)], idxv)

        def remap(j, c2):
            vv = idxv[pl.ds(j * 16, 16)]
            idxgs[buf][pl.ds(j * 16, 16)] = jnp.where(vv < _TAIL, vv,
                                                      2 * vv - _TAIL)
            return c2

        lax.fori_loop(0, 8, remap, 0, unroll=8)
        pltpu.async_copy(tbl_hbm.at[idxgs[buf]], rowss[buf], gsems[buf])

    stage(0, 0)

    def do_group(i, buf):
        g = wid * ngrp + i
        s = g // 128
        cb = g % 128
        pltpu.make_async_copy(tbl_hbm.at[idxgs[buf]], rowss[buf],
                              gsems[buf]).wait()

        @pl.when(i + 1 < ngrp)
        def _():
            stage(i + 1, 1 - buf)

        rows = rowss[buf]

        @plsc.parallel_loop(0, 128, unroll=4)
        def _(lb):
            col = jnp.full((16,), lb, dtype=jnp.int32)
            for dg in range(4):
                vec = rows[lb, pl.ds(dg * 16, 16)] * _SCALE
                plsc.store_scatter(outc, [dgrow[dg], col], vec)
        for gd in range(8):
            pltpu.async_copy(outc.at[pl.ds(gd * 8, 8), pl.ds(0, 128)],
                             out_hbm.at[s, gd, cb], wsem)
        for gd in range(8):
            pltpu.make_async_copy(outc.at[pl.ds(gd * 8, 8), pl.ds(0, 128)],
                                  out_hbm.at[s, gd, cb], wsem).wait()

    def grp_loop(i0, carry):
        for b in range(2):
            do_group(i0 * 2 + b, b)
        return carry

    lax.fori_loop(0, ngrp // 2, grp_loop, 0)


def kernel(x, emb_weight):
    natv = jnp.transpose(emb_weight)
    xf = jnp.reshape(jnp.transpose(x), (327680,)).astype(jnp.int32)
    tailp = jnp.concatenate(
        [emb_weight[_TAIL:, :],
         jnp.zeros((_V - _TAIL, _D), jnp.float32)], axis=1)

    mesh = plsc.VectorSubcoreMesh(core_axis_name="c", subcore_axis_name="s")
    k1 = pl.kernel(
        _transpose_body,
        mesh=mesh,
        out_type=jax.ShapeDtypeStruct((_SROWS, 128), jnp.float32),
        scratch_types=[
            pltpu.VMEM((64, 128), jnp.float32),
            pltpu.VMEM((64, 128), jnp.float32),
            pltpu.VMEM((64 * _SKEW,), jnp.float32),
            pltpu.VMEM((64, 128), jnp.float32),
            pltpu.VMEM((64, 128), jnp.float32),
            pltpu.SemaphoreType.DMA,
            pltpu.SemaphoreType.DMA,
            pltpu.SemaphoreType.DMA,
            pltpu.SemaphoreType.DMA,
        ],
        compiler_params=pltpu.CompilerParams(
            use_tc_tiling_on_sc=True, needs_layout_passes=False),
    )
    scratch = k1(natv, tailp)
    scr_view = jnp.reshape(scratch, (2 * _SROWS, _D))

    k2 = pl.kernel(
        _gather_body,
        mesh=mesh,
        out_type=jax.ShapeDtypeStruct((20, 8, 128, 8, 128), jnp.float32),
        scratch_types=[
            pltpu.VMEM((128,), jnp.int32),
            pltpu.VMEM((128,), jnp.int32),
            pltpu.VMEM((128,), jnp.int32),
            pltpu.VMEM((128, _D), jnp.float32),
            pltpu.VMEM((128, _D), jnp.float32),
            pltpu.VMEM((_D, 136), jnp.float32),
            pltpu.SemaphoreType.DMA,
            pltpu.SemaphoreType.DMA,
            pltpu.SemaphoreType.DMA,
        ],
        compiler_params=pltpu.CompilerParams(
            use_tc_tiling_on_sc=False, needs_layout_passes=False),
    )
    out5 = k2(scr_view, xf)
    return jnp.reshape(jnp.transpose(out5, (2, 4, 0, 1, 3)), (16384, 20, 64))


# speedup vs baseline: 6.2833x; 1.0043x over previous
import math

import jax
import jax.numpy as jnp
from jax import lax
from jax.experimental import pallas as pl
from jax.experimental.pallas import tpu as pltpu
from jax.experimental.pallas import tpu_sc as plsc

_V = 1000000
_D = 64
_SCALE = math.sqrt(_D)
_NC = 2
_NS = 16
_NW = _NC * _NS
_NFULL = _V // 128
_TAIL = _NFULL * 128
_SROWS = _TAIL // 2 + 64
_SKEW = 137


def _transpose_body(nat_hbm, tail_hbm, scr_hbm, inb0, inb1, tb1, tbc0, tbc1,
                    isem0, isem1, osem0, osem1):
    wid = lax.axis_index("s") * _NC + lax.axis_index("c")
    iota = lax.iota(jnp.int32, 16)
    vbase = [(vg * 8 + (iota >> 1)) * _SKEW + (iota & 1) * 72
             for vg in range(8)]
    inbs = (inb0, inb1)
    isems = (isem0, isem1)
    tbcs = (tbc0, tbc1)
    osems = (osem0, osem1)
    nt = (_NFULL + _NW - 1) // _NW

    def cof(t):
        return (wid + t * _NW) * 128

    pltpu.async_copy(nat_hbm.at[:, pl.ds(cof(0), 128)], inb0, isem0)

    def do_block(t, buf):
        c = wid + t * _NW
        inb = inbs[buf]

        @pl.when(c + _NW < _NFULL)
        def _():
            pltpu.async_copy(nat_hbm.at[:, pl.ds(cof(t + 1), 128)],
                             inbs[1 - buf], isems[1 - buf])

        pltpu.make_async_copy(nat_hbm.at[:, pl.ds(cof(t), 128)], inb,
                              isems[buf]).wait()

        @plsc.parallel_loop(0, 64, unroll=4)
        def _(d):
            for vg in range(8):
                vec = inb[d, pl.ds(vg * 16, 16)]
                plsc.store_scatter(tb1, [vbase[vg] + d], vec)

        @pl.when(t >= 2)
        def _():
            cp = c - 2 * _NW
            pltpu.make_async_copy(tbcs[buf],
                                  scr_hbm.at[pl.ds(cp * 64, 64)],
                                  osems[buf]).wait()

        @plsc.parallel_loop(0, 64, unroll=4)
        def _(q):
            for h in range(2):
                for dg in range(4):
                    tbcs[buf][q, pl.ds(h * 64 + dg * 16, 16)] = tb1[
                        pl.ds(q * _SKEW + h * 72 + dg * 16, 16)]
        pltpu.async_copy(tbcs[buf], scr_hbm.at[pl.ds(c * 64, 64)],
                         osems[buf])

    def blk_loop(t0, carry):
        for b in range(2):
            t = t0 * 2 + b

            @pl.when(wid + t * _NW < _NFULL)
            def _():
                do_block(t, b)

        return carry

    lax.fori_loop(0, (nt + 1) // 2, blk_loop, 0)

    tmax = (_NFULL - 1 - wid) // _NW
    c1 = wid + tmax * _NW
    for b in range(2):
        cb = jnp.where(tmax % 2 == b, c1, c1 - _NW)
        pltpu.make_async_copy(tbcs[b], scr_hbm.at[pl.ds(cb * 64, 64)],
                              osems[b]).wait()

    @pl.when(wid == 0)
    def _():
        pltpu.sync_copy(tail_hbm, inb0)
        pltpu.sync_copy(inb0, scr_hbm.at[pl.ds(_TAIL // 2, 64)])


def _gather_body(tbl_hbm, xf_hbm, out_hbm, idxv, idxg0, idxg1, rows0, rows1,
                 outc0, outc1, gsem0, gsem1, wsem0, wsem1):
    wid = lax.axis_index("s") * _NC + lax.axis_index("c")
    ngrp = 2560 // _NW
    iota = lax.iota(jnp.int32, 16)
    dgrow = [dg * 16 + iota for dg in range(4)]
    idxgs = (idxg0, idxg1)
    rowss = (rows0, rows1)
    gsems = (gsem0, gsem1)
    outcs = (outc0, outc1)
    wsems = (wsem0, wsem1)

    def wb_waits(g, buf):
        s = g // 128
        cb = g % 128
        for gd in range(8):
            pltpu.make_async_copy(
                outcs[buf].at[pl.ds(gd * 8, 8), pl.ds(0, 128)],
                out_hbm.at[s, gd, cb], wsems[buf]).wait()

    def stage(i, buf):
        g = wid * ngrp + i
        base = (g // 128) * 16384 + (g % 128) * 128
        pltpu.sync_copy(xf_hbm.at[pl.ds(base, 128)], idxv)

        def remap(j, c2):
            vv = idxv[pl.ds(j * 16, 16)]
            idxgs[buf][pl.ds(j * 16, 16)] = jnp.where(vv < _TAIL, vv,
                                                      2 * vv - _TAIL)
            return c2

        lax.fori_loop(0, 8, remap, 0, unroll=8)
        pltpu.async_copy(tbl_hbm.at[idxgs[buf]], rowss[buf], gsems[buf])

    stage(0, 0)

    def do_group(i, buf):
        g = wid * ngrp + i
        s = g // 128
        cb = g % 128
        pltpu.make_async_copy(tbl_hbm.at[idxgs[buf]], rowss[buf],
                              gsems[buf]).wait()

        @pl.when(i + 1 < ngrp)
        def _():
            stage(i + 1, 1 - buf)

        rows = rowss[buf]

        @pl.when(i >= 2)
        def _():
            wb_waits(g - 2, buf)

        @plsc.parallel_loop(0, 128, unroll=4)
        def _(lb):
            col = jnp.full((16,), lb, dtype=jnp.int32)
            for dg in range(4):
                vec = rows[lb, pl.ds(dg * 16, 16)] * _SCALE
                plsc.store_scatter(outcs[buf], [dgrow[dg], col], vec)
        for gd in range(8):
            pltpu.async_copy(outcs[buf].at[pl.ds(gd * 8, 8), pl.ds(0, 128)],
                             out_hbm.at[s, gd, cb], wsems[buf])

    def grp_loop(i0, carry):
        for b in range(2):
            do_group(i0 * 2 + b, b)
        return carry

    lax.fori_loop(0, ngrp // 2, grp_loop, 0)
    for b in range(2):
        wb_waits(wid * ngrp + ngrp - 2 + b, b)


def kernel(x, emb_weight):
    natv = jnp.transpose(emb_weight)
    xf = jnp.reshape(jnp.transpose(x), (327680,)).astype(jnp.int32)
    tailp = jnp.concatenate(
        [emb_weight[_TAIL:, :],
         jnp.zeros((_V - _TAIL, _D), jnp.float32)], axis=1)

    mesh = plsc.VectorSubcoreMesh(core_axis_name="c", subcore_axis_name="s")
    k1 = pl.kernel(
        _transpose_body,
        mesh=mesh,
        out_type=jax.ShapeDtypeStruct((_SROWS, 128), jnp.float32),
        scratch_types=[
            pltpu.VMEM((64, 128), jnp.float32),
            pltpu.VMEM((64, 128), jnp.float32),
            pltpu.VMEM((64 * _SKEW,), jnp.float32),
            pltpu.VMEM((64, 128), jnp.float32),
            pltpu.VMEM((64, 128), jnp.float32),
            pltpu.SemaphoreType.DMA,
            pltpu.SemaphoreType.DMA,
            pltpu.SemaphoreType.DMA,
            pltpu.SemaphoreType.DMA,
        ],
        compiler_params=pltpu.CompilerParams(
            use_tc_tiling_on_sc=True, needs_layout_passes=False),
    )
    scratch = k1(natv, tailp)
    scr_view = jnp.reshape(scratch, (2 * _SROWS, _D))

    k2 = pl.kernel(
        _gather_body,
        mesh=mesh,
        out_type=jax.ShapeDtypeStruct((20, 8, 128, 8, 128), jnp.float32),
        scratch_types=[
            pltpu.VMEM((128,), jnp.int32),
            pltpu.VMEM((128,), jnp.int32),
            pltpu.VMEM((128,), jnp.int32),
            pltpu.VMEM((128, _D), jnp.float32),
            pltpu.VMEM((128, _D), jnp.float32),
            pltpu.VMEM((_D, 136), jnp.float32),
            pltpu.VMEM((_D, 136), jnp.float32),
            pltpu.SemaphoreType.DMA,
            pltpu.SemaphoreType.DMA,
            pltpu.SemaphoreType.DMA,
            pltpu.SemaphoreType.DMA,
        ],
        compiler_params=pltpu.CompilerParams(
            use_tc_tiling_on_sc=False, needs_layout_passes=False),
    )
    out5 = k2(scr_view, xf)
    return jnp.reshape(jnp.transpose(out5, (2, 4, 0, 1, 3)), (16384, 20, 64))
